# baseline (speedup 1.0000x reference)
import functools

import jax
import jax.numpy as jnp
from jax import lax
from jax.experimental import pallas as pl
from jax.experimental.pallas import tpu as pltpu
from jax.experimental.pallas import tpu_sc as plsc

N_FILT = 150000
N_ORIG = 200000
NC, NS, L = 2, 16, 16
RB = 128
ROWS = 74
CA = ROWS * RB
CG = CA // NC
NBINS = 256
LAST = N_FILT - (NS - 1) * CA
LROWS = LAST // RB
LREM = LAST - LROWS * RB
LGB = LAST - CG
I32 = jnp.int32

_mesh = plsc.VectorSubcoreMesh(core_axis_name="c", subcore_axis_name="s")


@functools.partial(
    pl.kernel,
    out_type=jax.ShapeDtypeStruct((N_FILT,), jnp.float32),
    mesh=_mesh,
    scratch_types=[
        pltpu.VMEM((2 * CA,), jnp.int32),
        pltpu.VMEM((ROWS, RB), jnp.int32),
        pltpu.VMEM((CA,), jnp.int32),
        pltpu.VMEM((CA,), jnp.float32),
        pltpu.VMEM((CA,), jnp.int32),
        pltpu.VMEM((NBINS,), jnp.float32),
        pltpu.VMEM((NS * NBINS,), jnp.float32),
        pltpu.VMEM((NBINS,), jnp.float32),
        pltpu.VMEM((CG,), jnp.float32),
        pltpu.VMEM_SHARED((NS * NBINS,), jnp.float32),
        pltpu.SemaphoreType.DMA,
    ],
    compiler_params=pltpu.CompilerParams(needs_layout_passes=False),
)
def _oc_gather_energy(idx2_hbm, sid_hbm, tbl_hbm, corr_hbm, out_hbm,
                      idx2_v, idx_v, sid_v, corr_v, epk, acc_v, pall_v, s_v,
                      out_v, shared, sem):
    c = lax.axis_index("c")
    s = lax.axis_index("s")
    last = s == NS - 1
    base = s * I32(CA)

    base2 = base * I32(2)

    @pl.when(jnp.logical_not(last))
    def _():
        pltpu.sync_copy(idx2_hbm.at[pl.ds(base2, 2 * CA)], idx2_v)

    @pl.when(last)
    def _():
        pltpu.sync_copy(idx2_hbm.at[pl.ds(base2, 2 * LAST)],
                        idx2_v.at[pl.ds(0, 2 * LAST)])

    zeros = jnp.zeros((L,), jnp.float32)
    iota = lax.iota(jnp.int32, L)
    col0 = jnp.zeros((L,), jnp.int32)

    def fire(j, carry):
        ro = j * I32(RB)
        for v in range(RB // L):
            o = I32(v * L)
            idx_v[j, pl.ds(o, L)] = plsc.load_gather(
                idx2_v, [(iota + ro + o) * I32(2)])
        pltpu.make_async_copy(
            tbl_hbm.at[idx_v.at[j]], epk.at[pl.ds(ro, RB)], sem
        ).start()
        return carry

    @pl.when(jnp.logical_not(last))
    def _():
        lax.fori_loop(I32(0), I32(ROWS), fire, 0)

    @pl.when(last)
    def _():
        lax.fori_loop(I32(0), I32(LROWS), fire, 0)
        jl = I32(LROWS)
        ro = I32(LROWS * RB)
        for v in range(LREM // L):
            o = I32(v * L)
            idx_v[jl, pl.ds(o, L)] = plsc.load_gather(
                idx2_v, [(iota + ro + o) * I32(2)])
        idx_v[jl, pl.ds(I32(LREM), L)] = col0
        pltpu.make_async_copy(
            tbl_hbm.at[idx_v.at[jl]], epk.at[pl.ds(ro, RB)], sem
        ).start()

    @pl.when(jnp.logical_not(last))
    def _():
        pltpu.sync_copy(sid_hbm.at[pl.ds(base, CA)], sid_v)
        pltpu.sync_copy(corr_hbm.at[pl.ds(base, CA)], corr_v)

    @pl.when(last)
    def _():
        pltpu.sync_copy(sid_hbm.at[pl.ds(base, LAST)],
                        sid_v.at[pl.ds(0, LAST)])
        pltpu.sync_copy(corr_hbm.at[pl.ds(base, LAST)],
                        corr_v.at[pl.ds(0, LAST)])

    def zero_body(k, carry):
        acc_v[pl.ds(k * I32(L), L)] = zeros
        return carry

    lax.fori_loop(I32(0), I32(NBINS // L), zero_body, 0)

    @pl.when(jnp.logical_not(last))
    def _():
        pltpu.make_async_copy(tbl_hbm.at[pl.ds(0, CA)], epk, sem).wait()

    @pl.when(last)
    def _():
        pltpu.make_async_copy(tbl_hbm.at[pl.ds(0, (LROWS + 1) * RB)],
                              epk.at[pl.ds(0, (LROWS + 1) * RB)], sem).wait()

    ones = jnp.ones((L,), jnp.int32)

    def body(i, carry):
        o = i * I32(L)
        sg = sid_v[pl.ds(o, L)]
        ev = epk[pl.ds(o, L)]
        t = jnp.bitwise_and(ev, ones)
        e = plsc.bitcast(jnp.bitwise_and(ev, ~ones), jnp.float32)
        cf = corr_v[pl.ds(o, L)]
        val = jnp.where(t == 0, e * cf, zeros)
        plsc.addupdate_scatter(acc_v, [sg], val)
        return carry

    @pl.when(jnp.logical_not(last))
    def _():
        lax.fori_loop(I32(0), I32(CA // L), body, 0)

    @pl.when(last)
    def _():
        lax.fori_loop(I32(0), I32(LAST // L), body, 0)

    pltpu.sync_copy(acc_v, shared.at[pl.ds(s * I32(NBINS), NBINS)])
    plsc.subcore_barrier()
    pltpu.sync_copy(shared, pall_v)

    def red(k, carry):
        o = k * I32(L)
        acc = pall_v[pl.ds(o, L)]
        for r in range(1, NS):
            acc = acc + pall_v[pl.ds(I32(r * NBINS) + o, L)]
        s_v[pl.ds(o, L)] = acc
        return carry

    lax.fori_loop(I32(0), I32(NBINS // L), red, 0)

    gbase = c * I32(CG)
    shortgb = jnp.logical_and(last, c == 1)

    def gat(i, carry):
        o = i * I32(L)
        sg = sid_v[pl.ds(gbase + o, L)]
        out_v[pl.ds(o, L)] = plsc.load_gather(s_v, [sg])
        return carry

    @pl.when(jnp.logical_not(shortgb))
    def _():
        lax.fori_loop(I32(0), I32(CG // L), gat, 0)

    @pl.when(shortgb)
    def _():
        lax.fori_loop(I32(0), I32(LGB // L), gat, 0)

    @pl.when(jnp.logical_not(shortgb))
    def _():
        pltpu.sync_copy(out_v, out_hbm.at[pl.ds(base + gbase, CG)])

    @pl.when(shortgb)
    def _():
        pltpu.sync_copy(out_v.at[pl.ds(0, LGB)],
                        out_hbm.at[pl.ds(base + gbase, LGB)])


def kernel(pred_sid, pred_corr_factor, rechit_energy, no_noise_idx,
           pred_beta, is_track):
    del pred_beta
    idx2 = no_noise_idx.view(jnp.int32).reshape(2 * N_FILT)
    sid = pred_sid.reshape(N_FILT)
    corr = pred_corr_factor.reshape(N_FILT)
    energy = rechit_energy.reshape(N_ORIG)
    istrack = is_track.view(jnp.int32).reshape(N_ORIG, 2)[:, 0]
    tbl = jnp.bitwise_or(
        jnp.bitwise_and(energy.view(jnp.int32), jnp.int32(-2)), istrack)

    out = _oc_gather_energy(idx2, sid, tbl, corr)
    return out[:, None]


# speedup vs baseline: 5.0556x; 5.0556x over previous
import functools

import jax
import jax.numpy as jnp
from jax import lax
from jax.experimental import pallas as pl
from jax.experimental.pallas import tpu as pltpu
from jax.experimental.pallas import tpu_sc as plsc

N_FILT = 150000
N_ORIG = 200000
NC, NS, L = 2, 16, 16
RB = 128
ROWS = 74
HROWS = ROWS // 2
CA = ROWS * RB
HA = CA // 2
NP = NS * CA
CG = CA // NC
NBINS = 256
I32 = jnp.int32

_mesh = plsc.VectorSubcoreMesh(core_axis_name="c", subcore_axis_name="s")


@functools.partial(
    pl.kernel,
    out_type=jax.ShapeDtypeStruct((NP,), jnp.float32),
    mesh=_mesh,
    scratch_types=[
        pltpu.VMEM((ROWS, RB), jnp.int32),
        pltpu.VMEM((CA,), jnp.int32),
        pltpu.VMEM((CA,), jnp.float32),
        pltpu.VMEM((CA,), jnp.int32),
        pltpu.VMEM((NBINS,), jnp.float32),
        pltpu.VMEM((NS * NBINS,), jnp.float32),
        pltpu.VMEM((NBINS,), jnp.float32),
        pltpu.VMEM((CG,), jnp.float32),
        pltpu.VMEM_SHARED((NS * NBINS,), jnp.float32),
        pltpu.SemaphoreType.DMA,
        pltpu.SemaphoreType.DMA,
    ],
    compiler_params=pltpu.CompilerParams(needs_layout_passes=False),
)
def _oc_gather_energy(idx_hbm, sid_hbm, tbl_hbm, corr_hbm, out_hbm,
                      idx_v, sid_v, corr_v, epk, acc_v, pall_v, s_v, out_v,
                      shared, sem_a, sem_b):
    c = lax.axis_index("c")
    s = lax.axis_index("s")
    pltpu.sync_copy(idx_hbm.at[s], idx_v)

    def fire_a(j, carry):
        pltpu.make_async_copy(
            tbl_hbm.at[idx_v.at[j]], epk.at[pl.ds(j * I32(RB), RB)], sem_a
        ).start()
        return carry

    def fire_b(j, carry):
        pltpu.make_async_copy(
            tbl_hbm.at[idx_v.at[j]], epk.at[pl.ds(j * I32(RB), RB)], sem_b
        ).start()
        return carry

    lax.fori_loop(I32(0), I32(HROWS), fire_a, 0)
    lax.fori_loop(I32(HROWS), I32(ROWS), fire_b, 0)

    pltpu.sync_copy(sid_hbm.at[pl.ds(s * I32(CA), CA)], sid_v)
    pltpu.sync_copy(corr_hbm.at[pl.ds(s * I32(CA), CA)], corr_v)

    zeros = jnp.zeros((L,), jnp.float32)
    ones = jnp.ones((L,), jnp.int32)

    def zero_body(k, carry):
        acc_v[pl.ds(k * I32(L), L)] = zeros
        return carry

    lax.fori_loop(I32(0), I32(NBINS // L), zero_body, 0)

    def body(i, carry):
        o = i * I32(L)
        sg = sid_v[pl.ds(o, L)]
        ev = epk[pl.ds(o, L)]
        t = jnp.bitwise_and(ev, ones)
        e = plsc.bitcast(jnp.bitwise_and(ev, ~ones), jnp.float32)
        cf = corr_v[pl.ds(o, L)]
        val = jnp.where(t == 0, e * cf, zeros)
        plsc.addupdate_scatter(acc_v, [sg], val)
        return carry

    pltpu.make_async_copy(tbl_hbm.at[pl.ds(0, HA)],
                          epk.at[pl.ds(0, HA)], sem_a).wait()
    lax.fori_loop(I32(0), I32(HA // L), body, 0)
    pltpu.make_async_copy(tbl_hbm.at[pl.ds(0, HA)],
                          epk.at[pl.ds(HA, HA)], sem_b).wait()
    lax.fori_loop(I32(HA // L), I32(CA // L), body, 0)

    pltpu.sync_copy(acc_v, shared.at[pl.ds(s * I32(NBINS), NBINS)])
    plsc.subcore_barrier()
    pltpu.sync_copy(shared, pall_v)

    def red(k, carry):
        o = k * I32(L)
        acc = pall_v[pl.ds(o, L)]
        for r in range(1, NS):
            acc = acc + pall_v[pl.ds(I32(r * NBINS) + o, L)]
        s_v[pl.ds(o, L)] = acc
        return carry

    lax.fori_loop(I32(0), I32(NBINS // L), red, 0)

    gbase = c * I32(CG)

    def gat(i, carry):
        o = i * I32(L)
        sg = sid_v[pl.ds(gbase + o, L)]
        out_v[pl.ds(o, L)] = plsc.load_gather(s_v, [sg])
        return carry

    lax.fori_loop(I32(0), I32(CG // L), gat, 0)
    pltpu.sync_copy(out_v, out_hbm.at[pl.ds(s * I32(CA) + gbase, CG)])


def kernel(pred_sid, pred_corr_factor, rechit_energy, no_noise_idx,
           pred_beta, is_track):
    del pred_beta
    idx = no_noise_idx[:, 0].astype(jnp.int32)
    sid = pred_sid[:, 0].astype(jnp.int32)
    corr = pred_corr_factor[:, 0].astype(jnp.float32)
    energy = rechit_energy[:, 0].astype(jnp.float32)
    istrack = is_track[:, 0].astype(jnp.int32)
    tbl = jnp.bitwise_or(
        jnp.bitwise_and(energy.view(jnp.int32), jnp.int32(-2)), istrack)

    pad = NP - N_FILT
    idx_p = jnp.pad(idx, (0, pad)).reshape(NS, ROWS, RB)
    sid_p = jnp.pad(sid, (0, pad))
    corr_p = jnp.pad(corr, (0, pad))

    out = _oc_gather_energy(idx_p, sid_p, tbl, corr_p)
    return out[:N_FILT, None]


# speedup vs baseline: 5.6104x; 1.1097x over previous
import functools

import jax
import jax.numpy as jnp
from jax import lax
from jax.experimental import pallas as pl
from jax.experimental.pallas import tpu as pltpu
from jax.experimental.pallas import tpu_sc as plsc

N_FILT = 150000
N_ORIG = 200000
NC, NS, L = 2, 16, 16
RB = 128
ROWS = 74
HROWS = ROWS // 2
CA = ROWS * RB
HA = CA // 2
NP = NS * CA
CG = CA // NC
NBINS = 256
I32 = jnp.int32

_mesh = plsc.VectorSubcoreMesh(core_axis_name="c", subcore_axis_name="s")


@functools.partial(
    pl.kernel,
    out_type=jax.ShapeDtypeStruct((NP,), jnp.float32),
    mesh=_mesh,
    scratch_types=[
        pltpu.VMEM((ROWS, RB), jnp.int32),
        pltpu.VMEM((CA,), jnp.int32),
        pltpu.VMEM((CA,), jnp.int32),
        pltpu.VMEM((NBINS,), jnp.float32),
        pltpu.VMEM((NS * NBINS,), jnp.float32),
        pltpu.VMEM((NBINS,), jnp.float32),
        pltpu.VMEM((CG,), jnp.float32),
        pltpu.VMEM_SHARED((NS * NBINS,), jnp.float32),
        pltpu.SemaphoreType.DMA,
        pltpu.SemaphoreType.DMA,
    ],
    compiler_params=pltpu.CompilerParams(needs_layout_passes=False),
)
def _oc_gather_energy(idx_hbm, skc_hbm, tbl_hbm, out_hbm,
                      idx_v, skc_v, epk, acc_v, pall_v, s_v, out_v,
                      shared, sem_a, sem_b):
    c = lax.axis_index("c")
    s = lax.axis_index("s")
    pltpu.sync_copy(idx_hbm.at[s], idx_v)

    def fire_a(j, carry):
        pltpu.make_async_copy(
            tbl_hbm.at[idx_v.at[j]], epk.at[pl.ds(j * I32(RB), RB)], sem_a
        ).start()
        return carry

    def fire_b(j, carry):
        pltpu.make_async_copy(
            tbl_hbm.at[idx_v.at[j]], epk.at[pl.ds(j * I32(RB), RB)], sem_b
        ).start()
        return carry

    lax.fori_loop(I32(0), I32(HROWS), fire_a, 0)
    lax.fori_loop(I32(HROWS), I32(ROWS), fire_b, 0)

    pltpu.sync_copy(skc_hbm.at[pl.ds(s * I32(CA), CA)], skc_v)

    zeros = jnp.zeros((L,), jnp.float32)
    ones = jnp.ones((L,), jnp.int32)
    m255 = jnp.full((L,), 255, jnp.int32)

    def zero_body(k, carry):
        acc_v[pl.ds(k * I32(L), L)] = zeros
        return carry

    lax.fori_loop(I32(0), I32(NBINS // L), zero_body, 0)

    def body(i, carry):
        o = i * I32(L)
        pk = skc_v[pl.ds(o, L)]
        sg = jnp.bitwise_and(pk, m255)
        cf = plsc.bitcast(jnp.bitwise_and(pk, ~m255), jnp.float32)
        ev = epk[pl.ds(o, L)]
        t = jnp.bitwise_and(ev, ones)
        e = plsc.bitcast(jnp.bitwise_and(ev, ~ones), jnp.float32)
        val = jnp.where(t == 0, e * cf, zeros)
        plsc.addupdate_scatter(acc_v, [sg], val)
        return carry

    pltpu.make_async_copy(tbl_hbm.at[pl.ds(0, HA)],
                          epk.at[pl.ds(0, HA)], sem_a).wait()
    lax.fori_loop(I32(0), I32(HA // L), body, 0)
    pltpu.make_async_copy(tbl_hbm.at[pl.ds(0, HA)],
                          epk.at[pl.ds(HA, HA)], sem_b).wait()
    lax.fori_loop(I32(HA // L), I32(CA // L), body, 0)

    pltpu.sync_copy(acc_v, shared.at[pl.ds(s * I32(NBINS), NBINS)])
    plsc.subcore_barrier()
    pltpu.sync_copy(shared, pall_v)

    def red(k, carry):
        o = k * I32(L)
        acc = pall_v[pl.ds(o, L)]
        for r in range(1, NS):
            acc = acc + pall_v[pl.ds(I32(r * NBINS) + o, L)]
        s_v[pl.ds(o, L)] = acc
        return carry

    lax.fori_loop(I32(0), I32(NBINS // L), red, 0)

    gbase = c * I32(CG)

    def gat(i, carry):
        o = i * I32(L)
        sg = jnp.bitwise_and(skc_v[pl.ds(gbase + o, L)], m255)
        out_v[pl.ds(o, L)] = plsc.load_gather(s_v, [sg])
        return carry

    lax.fori_loop(I32(0), I32(CG // L), gat, 0)
    pltpu.sync_copy(out_v, out_hbm.at[pl.ds(s * I32(CA) + gbase, CG)])


def kernel(pred_sid, pred_corr_factor, rechit_energy, no_noise_idx,
           pred_beta, is_track):
    del pred_beta
    idx = no_noise_idx[:, 0].astype(jnp.int32)
    energy = rechit_energy[:, 0].astype(jnp.float32)
    istrack = is_track[:, 0].astype(jnp.int32)
    tbl = jnp.bitwise_or(
        jnp.bitwise_and(energy.view(jnp.int32), jnp.int32(-2)), istrack)
    skc = jnp.bitwise_or(
        jnp.bitwise_and(pred_corr_factor.view(jnp.int32), jnp.int32(-256)),
        pred_sid)[:, 0]

    pad = NP - N_FILT
    idx_p = jnp.pad(idx, (0, pad)).reshape(NS, ROWS, RB)
    skc_p = jnp.pad(skc, (0, pad))

    out = _oc_gather_energy(idx_p, skc_p, tbl)
    return out[:N_FILT, None]


# speedup vs baseline: 5.6765x; 1.0118x over previous
import functools

import jax
import jax.numpy as jnp
from jax import lax
from jax.experimental import pallas as pl
from jax.experimental.pallas import tpu as pltpu
from jax.experimental.pallas import tpu_sc as plsc

N_FILT = 150000
N_ORIG = 200000
NC, NS, L = 2, 16, 16
RB = 128
ROWS = 74
HROWS = ROWS // 2
CA = ROWS * RB
HA = CA // 2
NP = NS * CA
CG = CA // NC
NBINS = 256
LGB = N_FILT - (NS - 1) * CA - CG
I32 = jnp.int32

_mesh = plsc.VectorSubcoreMesh(core_axis_name="c", subcore_axis_name="s")


@functools.partial(
    pl.kernel,
    out_type=jax.ShapeDtypeStruct((N_FILT,), jnp.float32),
    mesh=_mesh,
    scratch_types=[
        pltpu.VMEM((ROWS, RB), jnp.int32),
        pltpu.VMEM((CA,), jnp.int32),
        pltpu.VMEM((CA,), jnp.int32),
        pltpu.VMEM((NBINS,), jnp.float32),
        pltpu.VMEM((NS * NBINS,), jnp.float32),
        pltpu.VMEM((NBINS,), jnp.float32),
        pltpu.VMEM((CG,), jnp.float32),
        pltpu.VMEM_SHARED((NS * NBINS,), jnp.float32),
        pltpu.SemaphoreType.DMA,
        pltpu.SemaphoreType.DMA,
    ],
    compiler_params=pltpu.CompilerParams(needs_layout_passes=False),
)
def _oc_gather_energy(idx_hbm, skc_hbm, tbl_hbm, out_hbm,
                      idx_v, skc_v, epk, acc_v, pall_v, s_v, out_v,
                      shared, sem_a, sem_b):
    c = lax.axis_index("c")
    s = lax.axis_index("s")
    pltpu.sync_copy(idx_hbm.at[s], idx_v)

    def fire_a(j, carry):
        pltpu.make_async_copy(
            tbl_hbm.at[idx_v.at[j]], epk.at[pl.ds(j * I32(RB), RB)], sem_a
        ).start()
        return carry

    def fire_b(j, carry):
        pltpu.make_async_copy(
            tbl_hbm.at[idx_v.at[j]], epk.at[pl.ds(j * I32(RB), RB)], sem_b
        ).start()
        return carry

    lax.fori_loop(I32(0), I32(HROWS), fire_a, 0)
    lax.fori_loop(I32(HROWS), I32(ROWS), fire_b, 0)

    pltpu.sync_copy(skc_hbm.at[pl.ds(s * I32(CA), CA)], skc_v)

    zeros = jnp.zeros((L,), jnp.float32)
    ones = jnp.ones((L,), jnp.int32)
    m255 = jnp.full((L,), 255, jnp.int32)

    def zero_body(k, carry):
        acc_v[pl.ds(k * I32(L), L)] = zeros
        return carry

    lax.fori_loop(I32(0), I32(NBINS // L), zero_body, 0)

    def body(i, carry):
        o = i * I32(L)
        pk = skc_v[pl.ds(o, L)]
        sg = jnp.bitwise_and(pk, m255)
        cf = plsc.bitcast(jnp.bitwise_and(pk, ~m255), jnp.float32)
        ev = epk[pl.ds(o, L)]
        t = jnp.bitwise_and(ev, ones)
        e = plsc.bitcast(jnp.bitwise_and(ev, ~ones), jnp.float32)
        val = jnp.where(t == 0, e * cf, zeros)
        plsc.addupdate_scatter(acc_v, [sg], val)
        return carry

    pltpu.make_async_copy(tbl_hbm.at[pl.ds(0, HA)],
                          epk.at[pl.ds(0, HA)], sem_a).wait()
    lax.fori_loop(I32(0), I32(HA // L), body, 0)
    pltpu.make_async_copy(tbl_hbm.at[pl.ds(0, HA)],
                          epk.at[pl.ds(HA, HA)], sem_b).wait()
    lax.fori_loop(I32(HA // L), I32(CA // L), body, 0)

    pltpu.sync_copy(acc_v, shared.at[pl.ds(s * I32(NBINS), NBINS)])
    plsc.subcore_barrier()
    pltpu.sync_copy(shared, pall_v)

    def red(k, carry):
        o = k * I32(L)
        acc = pall_v[pl.ds(o, L)]
        for r in range(1, NS):
            acc = acc + pall_v[pl.ds(I32(r * NBINS) + o, L)]
        s_v[pl.ds(o, L)] = acc
        return carry

    lax.fori_loop(I32(0), I32(NBINS // L), red, 0)

    gbase = c * I32(CG)

    def gat(i, carry):
        o = i * I32(L)
        sg = jnp.bitwise_and(skc_v[pl.ds(gbase + o, L)], m255)
        out_v[pl.ds(o, L)] = plsc.load_gather(s_v, [sg])
        return carry

    lax.fori_loop(I32(0), I32(CG // L), gat, 0)
    obase = s * I32(CA) + gbase
    shortw = jnp.logical_and(s == NS - 1, c == 1)

    @pl.when(jnp.logical_not(shortw))
    def _():
        pltpu.sync_copy(out_v, out_hbm.at[pl.ds(obase, CG)])

    @pl.when(shortw)
    def _():
        pltpu.sync_copy(out_v.at[pl.ds(0, LGB)], out_hbm.at[pl.ds(obase, LGB)])


def kernel(pred_sid, pred_corr_factor, rechit_energy, no_noise_idx,
           pred_beta, is_track):
    del pred_beta
    idx = no_noise_idx[:, 0].astype(jnp.int32)
    energy = rechit_energy[:, 0].astype(jnp.float32)
    istrack = is_track[:, 0].astype(jnp.int32)
    tbl = jnp.bitwise_or(
        jnp.bitwise_and(energy.view(jnp.int32), jnp.int32(-2)), istrack)
    skc = jnp.bitwise_or(
        jnp.bitwise_and(pred_corr_factor.view(jnp.int32), jnp.int32(-256)),
        pred_sid)[:, 0]

    pad = NP - N_FILT
    idx_p = jnp.pad(idx, (0, pad)).reshape(NS, ROWS, RB)
    skc_p = jnp.pad(skc, (0, pad))

    out = _oc_gather_energy(idx_p, skc_p, tbl)
    return out[:, None]
